# baseline (speedup 1.0000x reference)
import functools

import jax
import jax.numpy as jnp
from jax import lax
from jax.experimental import pallas as pl
from jax.experimental.pallas import tpu as pltpu
from jax.experimental.pallas import tpu_sc as plsc

B = 16384
D = 64
DP = 128
NC = 2
NS = 16
L = 16
NW = NC * NS
BPW = B // NW
CHUNK = 128
NCH = BPW // CHUNK
NBUF = 2


def _fm_body(inp_hbm, ut_hbm, it_hbm, ub_hbm, ib_hbm, out_hbm,
             inp_v, uidx_v, iidx_v, urows_v, irows_v, ubias_v, ibias_v,
             out_v, sem):
    wid = lax.axis_index("s") * NC + lax.axis_index("c")
    base = wid * BPW
    lane = lax.iota(jnp.int32, L)

    pltpu.sync_copy(inp_hbm.at[pl.ds(base * 2, 2 * BPW)], inp_v)
    for j in range(BPW // L):
        offs = lane * 2 + (2 * L) * j
        uidx_v[j // (CHUNK // L), pl.ds((j % (CHUNK // L)) * L, L)] = (
            plsc.load_gather(inp_v, [offs]))
        iidx_v[j // (CHUNK // L), pl.ds((j % (CHUNK // L)) * L, L)] = (
            plsc.load_gather(inp_v, [offs + 1]))

    def fire(j):
        buf = j % NBUF
        return [
            pltpu.async_copy(ut_hbm.at[uidx_v.at[j]], urows_v.at[buf], sem),
            pltpu.async_copy(it_hbm.at[iidx_v.at[j]], irows_v.at[buf], sem),
            pltpu.async_copy(ub_hbm.at[uidx_v.at[j]],
                             ubias_v.at[pl.ds(j * CHUNK, CHUNK)], sem),
            pltpu.async_copy(ib_hbm.at[iidx_v.at[j]],
                             ibias_v.at[pl.ds(j * CHUNK, CHUNK)], sem),
        ]

    pending = fire(0)
    for j in range(NCH):
        for c in pending:
            c.wait()
        if j + 1 < NCH:
            pending = fire(j + 1)
        buf = j % NBUF

        for g in range(CHUNK // L):
            dots = jnp.zeros((L,), jnp.float32)
            for r in range(L):
                row = g * L + r
                s = (urows_v[buf, row, pl.ds(0, L)] *
                     irows_v[buf, row, pl.ds(0, L)])
                for c in range(1, D // L):
                    s = s + (urows_v[buf, row, pl.ds(c * L, L)] *
                             irows_v[buf, row, pl.ds(c * L, L)])
                dots = jnp.where(lane == r, jnp.sum(s), dots)
            blk = pl.ds(j * CHUNK + g * L, L)
            out_v[blk] = dots + ubias_v[blk] + ibias_v[blk]

    pltpu.sync_copy(out_v, out_hbm.at[pl.ds(base, BPW)])


@functools.partial(
    pl.kernel,
    out_type=jax.ShapeDtypeStruct((B,), jnp.float32),
    mesh=plsc.VectorSubcoreMesh(core_axis_name="c", subcore_axis_name="s"),
    compiler_params=pltpu.CompilerParams(needs_layout_passes=False),
    scratch_types=[
        pltpu.VMEM((2 * BPW,), jnp.int32),
        pltpu.VMEM((NCH, CHUNK), jnp.int32),
        pltpu.VMEM((NCH, CHUNK), jnp.int32),
        pltpu.VMEM((NBUF, CHUNK, DP), jnp.float32),
        pltpu.VMEM((NBUF, CHUNK, DP), jnp.float32),
        pltpu.VMEM((BPW,), jnp.float32),
        pltpu.VMEM((BPW,), jnp.float32),
        pltpu.VMEM((BPW,), jnp.float32),
        pltpu.SemaphoreType.DMA,
    ],
)
def _fm(inp_hbm, ut_hbm, it_hbm, ub_hbm, ib_hbm, out_hbm, *scratch):
    _fm_body(inp_hbm, ut_hbm, it_hbm, ub_hbm, ib_hbm, out_hbm, *scratch)


def kernel(inputs, user_emb_table, item_emb_table, user_bias_table, item_bias_table):
    flat_idx = inputs.astype(jnp.int32).reshape(-1)
    up = jnp.pad(user_emb_table, ((0, 0), (0, DP - D)))
    ip = jnp.pad(item_emb_table, ((0, 0), (0, DP - D)))
    out = _fm(flat_idx, up, ip,
              user_bias_table.reshape(-1), item_bias_table.reshape(-1))
    return out.reshape(B, 1)
